# baseline (speedup 1.0000x reference)
import functools

import jax
import jax.numpy as jnp
from jax import lax
from jax.experimental import pallas as pl
from jax.experimental.pallas import tpu as pltpu
from jax.experimental.pallas import tpu_sc as plsc

VOCAB = 100000
EMB = 64
BATCH = 4096
FIELDS = 26
FPAD = 32
RPAD = BATCH * FPAD
OUT2_ROWS = RPAD // 2

NC = 2
NS = 16
NW = NC * NS
R_PER_W = RPAD // NW
NPAD = 256
PCH = NPAD // 2
NCHUNK = R_PER_W // NPAD
NBUF = 2

_mesh = plsc.VectorSubcoreMesh(core_axis_name="c", subcore_axis_name="s")


@functools.partial(
    pl.kernel,
    mesh=_mesh,
    out_type=jax.ShapeDtypeStruct((OUT2_ROWS, 128), jnp.float32),
    compiler_params=pltpu.CompilerParams(use_tc_tiling_on_sc=False),
    scratch_types=[
        pltpu.VMEM((R_PER_W,), jnp.int32),
        pltpu.VMEM((R_PER_W + 16,), jnp.int32),
        pltpu.VMEM((NBUF, NPAD, 128), jnp.float32),
        pltpu.VMEM((NBUF, PCH, 128), jnp.float32),
        pltpu.SemaphoreType.DMA((NBUF,)),
        pltpu.SemaphoreType.DMA((NBUF,)),
    ],
)
def _emb_gather(pidx_hbm, hb_hbm, table2_hbm, out2_hbm, pidx_v, hb_v,
                g_v, o_v, gsem, ssem):
    wid = lax.axis_index("s") * NC + lax.axis_index("c")
    rbase = wid * R_PER_W
    obase = rbase // 2
    pltpu.sync_copy(pidx_hbm.at[pl.ds(rbase, R_PER_W)], pidx_v)
    pltpu.sync_copy(
        hb_hbm.at[pl.ds(rbase, R_PER_W)], hb_v.at[pl.ds(0, R_PER_W)]
    )

    def gather(ci, b):
        return pltpu.async_copy(
            table2_hbm.at[pidx_v.at[pl.ds(ci * NPAD, NPAD)]],
            g_v.at[b],
            gsem.at[b],
        )

    def select(ci, b):
        def body(r, _):
            h = hb_v[pl.ds(ci * NPAD + r, 16)][0]
            p = r // 2
            c0 = (r % 2) * 64
            for gi in range(4):
                o_v[b, p, pl.ds(c0 + gi * 16, 16)] = g_v[
                    b, r, pl.ds(h + gi * 16, 16)
                ]
            return 0

        lax.fori_loop(0, NPAD, body, 0)

    g = {}
    s = {}
    for ci in range(NBUF):
        g[ci] = gather(ci, ci)
    for ci in range(NCHUNK):
        b = ci % NBUF
        g[ci].wait()
        if ci >= NBUF:
            s[ci - NBUF].wait()
        select(ci, b)
        s[ci] = pltpu.async_copy(
            o_v.at[b], out2_hbm.at[pl.ds(obase + ci * PCH, PCH)], ssem.at[b]
        )
        nx = ci + NBUF
        if nx < NCHUNK:
            g[nx] = gather(nx, b)
    for ci in range(NCHUNK - NBUF, NCHUNK):
        s[ci].wait()


def kernel(x, table):
    idxp = jnp.pad(x.astype(jnp.int32), ((0, 0), (0, FPAD - FIELDS))).reshape(
        RPAD
    )
    pidx = idxp >> 1
    hb = (idxp & 1) << 6
    table2 = table.reshape(VOCAB // 2, 128)
    out2 = _emb_gather(pidx, hb, table2)
    return out2.reshape(BATCH, FPAD, EMB)[:, :FIELDS, :]


# speedup vs baseline: 6.8739x; 6.8739x over previous
import functools

import jax
import jax.numpy as jnp
from jax import lax
from jax.experimental import pallas as pl
from jax.experimental.pallas import tpu as pltpu
from jax.experimental.pallas import tpu_sc as plsc

VOCAB = 100000
EMB = 64
BATCH = 4096
FIELDS = 26
BFLAT = BATCH * FIELDS

NC = 2
NS = 16
NW = NC * NS
B_PER_W = BFLAT // NW
CHUNK = 416
NCHUNK = B_PER_W // CHUNK
NBUF = 4

_mesh = plsc.VectorSubcoreMesh(core_axis_name="c", subcore_axis_name="s")


@functools.partial(
    pl.kernel,
    mesh=_mesh,
    out_type=jax.ShapeDtypeStruct((BFLAT, EMB), jnp.float32),
    compiler_params=pltpu.CompilerParams(use_tc_tiling_on_sc=False),
    scratch_types=[
        pltpu.VMEM((B_PER_W,), jnp.int32),
        pltpu.VMEM((NBUF, CHUNK, EMB), jnp.float32),
        pltpu.SemaphoreType.DMA((NBUF,)),
        pltpu.SemaphoreType.DMA((NBUF,)),
    ],
)
def _emb_gather(idx_hbm, table_hbm, out_hbm, idx_v, rows_v, gsem, ssem):
    wid = lax.axis_index("s") * NC + lax.axis_index("c")
    base = wid * B_PER_W
    pltpu.sync_copy(idx_hbm.at[pl.ds(base, B_PER_W)], idx_v)

    def gather(ci, b):
        return pltpu.async_copy(
            table_hbm.at[idx_v.at[pl.ds(ci * CHUNK, CHUNK)]],
            rows_v.at[b],
            gsem.at[b],
        )

    g = {}
    s = {}
    for ci in range(NBUF):
        g[ci] = gather(ci, ci)
    for ci in range(NCHUNK):
        b = ci % NBUF
        g[ci].wait()
        s[ci] = pltpu.async_copy(
            rows_v.at[b], out_hbm.at[pl.ds(base + ci * CHUNK, CHUNK)], ssem.at[b]
        )
        nx = ci + NBUF
        if nx < NCHUNK:
            s[ci].wait()
            g[nx] = gather(nx, b)
    for ci in range(max(0, NCHUNK - NBUF), NCHUNK):
        s[ci].wait()


def kernel(x, table):
    idx = x.reshape(BFLAT).astype(jnp.int32)
    out = _emb_gather(idx, table)
    return out.reshape(BATCH, FIELDS, EMB)


# speedup vs baseline: 6.9747x; 1.0147x over previous
import functools

import jax
import jax.numpy as jnp
from jax import lax
from jax.experimental import pallas as pl
from jax.experimental.pallas import tpu as pltpu
from jax.experimental.pallas import tpu_sc as plsc

VOCAB = 100000
EMB = 64
BATCH = 4096
FIELDS = 26

NC = 2
NS = 16
NW = NC * NS
E_PER_W = EMB // NW
GROUPS = BATCH // 16

_mesh = plsc.VectorSubcoreMesh(core_axis_name="c", subcore_axis_name="s")


@functools.partial(
    pl.kernel,
    mesh=_mesh,
    out_type=jax.ShapeDtypeStruct((FIELDS, EMB, BATCH), jnp.float32),
    compiler_params=pltpu.CompilerParams(
        use_tc_tiling_on_sc=False, needs_layout_passes=False
    ),
    scratch_types=[
        pltpu.VMEM((VOCAB,), jnp.float32),
        pltpu.VMEM((2, BATCH), jnp.int32),
        pltpu.VMEM((2, BATCH), jnp.float32),
        pltpu.VMEM_SHARED((FIELDS, BATCH), jnp.int32),
        pltpu.SemaphoreType.DMA((2,)),
        pltpu.SemaphoreType.DMA((2,)),
    ],
)
def _emb_gather(xT_hbm, tableT_hbm, outT_hbm, erow_v, xr_v, ob_v, xsh,
                xsem, osem):
    cid = lax.axis_index("c")
    sid = lax.axis_index("s")
    wid = sid * NC + cid

    @pl.when(sid == 0)
    def _stage_x():
        pltpu.sync_copy(xT_hbm, xsh)

    plsc.subcore_barrier()

    def xwait(slot):
        pltpu.make_async_copy(
            xT_hbm.at[0], xr_v.at[slot], xsem.at[slot]
        ).wait()

    def owait(slot):
        pltpu.make_async_copy(
            outT_hbm.at[0, 0], ob_v.at[slot], osem.at[slot]
        ).wait()

    for es in range(E_PER_W):
        e = wid * E_PER_W + es
        pltpu.sync_copy(tableT_hbm.at[e], erow_v)
        pltpu.async_copy(xsh.at[0], xr_v.at[0], xsem.at[0])
        pltpu.async_copy(xsh.at[1], xr_v.at[1], xsem.at[1])

        def fbody(f2, carry, es=es, e=e):
            for slot in range(2):
                f = f2 * 2 + slot
                xwait(slot)
                if es == 0:
                    @pl.when(f2 >= 1)
                    def _():
                        owait(slot)
                else:
                    owait(slot)
                for g in range(GROUPS):
                    idx = xr_v[slot, pl.ds(g * 16, 16)]
                    ob_v[slot, pl.ds(g * 16, 16)] = plsc.load_gather(
                        erow_v, [idx]
                    )
                pltpu.async_copy(
                    ob_v.at[slot], outT_hbm.at[f, e], osem.at[slot]
                )

                @pl.when(f2 < (FIELDS // 2) - 1)
                def _():
                    pltpu.async_copy(
                        xsh.at[f + 2], xr_v.at[slot], xsem.at[slot]
                    )

            return carry

        lax.fori_loop(0, FIELDS // 2, fbody, 0)
    owait(0)
    owait(1)


def kernel(x, table):
    outT = _emb_gather(x.T, table.T)
    return outT.transpose(2, 0, 1)


# speedup vs baseline: 7.6318x; 1.0942x over previous
import functools

import jax
import jax.numpy as jnp
from jax import lax
from jax.experimental import pallas as pl
from jax.experimental.pallas import tpu as pltpu
from jax.experimental.pallas import tpu_sc as plsc

VOCAB = 100000
EMB = 64
BATCH = 4096
FIELDS = 26

NC = 2
NS = 16
NW = NC * NS
E_PER_W = EMB // NW
GROUPS = BATCH // 16
XRING = 4

_mesh = plsc.VectorSubcoreMesh(core_axis_name="c", subcore_axis_name="s")


@functools.partial(
    pl.kernel,
    mesh=_mesh,
    out_type=jax.ShapeDtypeStruct((FIELDS, 8, 32, 8, 128), jnp.float32),
    compiler_params=pltpu.CompilerParams(
        use_tc_tiling_on_sc=False, needs_layout_passes=False
    ),
    scratch_types=[
        pltpu.VMEM((VOCAB,), jnp.float32),
        pltpu.VMEM((XRING, BATCH), jnp.int32),
        pltpu.VMEM((2, 32, 128), jnp.float32),
        pltpu.SemaphoreType.DMA((XRING,)),
        pltpu.SemaphoreType.DMA((2,)),
    ],
)
def _emb_gather(xT_hbm, tableT_hbm, outQ_hbm, erow_v, xr_v, ob_v,
                xsem, osem):
    cid = lax.axis_index("c")
    sid = lax.axis_index("s")
    wid = sid * NC + cid

    def xwait(slot):
        pltpu.make_async_copy(
            xT_hbm.at[0], xr_v.at[slot], xsem.at[slot]
        ).wait()

    def owait(slot):
        pltpu.make_async_copy(
            outQ_hbm.at[0, 0, :, 0], ob_v.at[slot], osem.at[slot]
        ).wait()

    def do_field(f, e, eb, k, xslot, oslot, first_store):
        xwait(xslot)
        if first_store is None:
            owait(oslot)
        else:
            @pl.when(first_store)
            def _():
                owait(oslot)
        for g in range(GROUPS):
            idx = xr_v[xslot, pl.ds(g * 16, 16)]
            ob_v[oslot, g // 8, pl.ds((g % 8) * 16, 16)] = plsc.load_gather(
                erow_v, [idx]
            )
        pltpu.async_copy(
            ob_v.at[oslot], outQ_hbm.at[f, eb, :, k], osem.at[oslot]
        )

        @pl.when(f < FIELDS - XRING)
        def _():
            pltpu.async_copy(
                xT_hbm.at[f + XRING], xr_v.at[xslot], xsem.at[xslot]
            )

    def ebody(es, carry):
        e = wid * E_PER_W + es
        eb = e // 8
        k = e % 8
        pltpu.sync_copy(tableT_hbm.at[e], erow_v)
        for slot in range(XRING):
            pltpu.async_copy(xT_hbm.at[slot], xr_v.at[slot], xsem.at[slot])

        def fbody(f4, carry2):
            for sub in range(XRING):
                f = f4 * XRING + sub
                oslot = sub % 2
                pred = (
                    jnp.logical_or(es > 0, f4 >= 1) if sub < 2 else None
                )
                do_field(f, e, eb, k, sub, oslot, pred)
            return carry2

        lax.fori_loop(0, FIELDS // XRING, fbody, 0)
        for sub in range(FIELDS % XRING):
            f_tail = FIELDS - (FIELDS % XRING) + sub
            do_field(f_tail, e, eb, k, sub, sub % 2, None)
        return carry

    lax.fori_loop(0, E_PER_W, ebody, 0)
    owait(0)
    owait(1)


def kernel(x, table):
    outQ = _emb_gather(x.T, table.T)
    return outQ.transpose(2, 4, 0, 1, 3).reshape(BATCH, FIELDS, EMB)


# speedup vs baseline: 9.8690x; 1.2932x over previous
import functools

import jax
import jax.numpy as jnp
from jax import lax
from jax.experimental import pallas as pl
from jax.experimental.pallas import tpu as pltpu
from jax.experimental.pallas import tpu_sc as plsc

VOCAB = 100000
EMB = 64
BATCH = 4096
FIELDS = 26

NC = 2
NS = 16
NW = NC * NS
E_PER_W = EMB // NW
GROUPS = BATCH // 16
XRING = 4

_mesh = plsc.VectorSubcoreMesh(core_axis_name="c", subcore_axis_name="s")


@functools.partial(
    pl.kernel,
    mesh=_mesh,
    out_type=jax.ShapeDtypeStruct((FIELDS, 8, 32, 8, 128), jnp.float32),
    compiler_params=pltpu.CompilerParams(
        use_tc_tiling_on_sc=False, needs_layout_passes=False
    ),
    scratch_types=[
        pltpu.VMEM((VOCAB,), jnp.float32),
        pltpu.VMEM((XRING, BATCH), jnp.int32),
        pltpu.VMEM((2, 32, 128), jnp.float32),
        pltpu.SemaphoreType.DMA((XRING,)),
        pltpu.SemaphoreType.DMA((2,)),
    ],
)
def _emb_gather(xT_hbm, tableT_hbm, outQ_hbm, erow_v, xr_v, ob_v,
                xsem, osem):
    cid = lax.axis_index("c")
    sid = lax.axis_index("s")
    wid = sid * NC + cid

    def xwait(slot):
        pltpu.make_async_copy(
            xT_hbm.at[0], xr_v.at[slot], xsem.at[slot]
        ).wait()

    def owait(slot):
        pltpu.make_async_copy(
            outQ_hbm.at[0, 0, :, 0], ob_v.at[slot], osem.at[slot]
        ).wait()

    def do_field(f, e, eb, k, xslot, oslot, first_store):
        xwait(xslot)
        if first_store is None:
            owait(oslot)
        else:
            @pl.when(first_store)
            def _():
                owait(oslot)
        U = 8
        for g0 in range(0, GROUPS, U):
            idxs = [
                xr_v[xslot, pl.ds((g0 + u) * 16, 16)] for u in range(U)
            ]
            vals = [plsc.load_gather(erow_v, [idxs[u]]) for u in range(U)]
            for u in range(U):
                g = g0 + u
                ob_v[oslot, g // 8, pl.ds((g % 8) * 16, 16)] = vals[u]
        pltpu.async_copy(
            ob_v.at[oslot], outQ_hbm.at[f, eb, :, k], osem.at[oslot]
        )

        @pl.when(f < FIELDS - XRING)
        def _():
            pltpu.async_copy(
                xT_hbm.at[f + XRING], xr_v.at[xslot], xsem.at[xslot]
            )

    def ebody(es, carry):
        e = wid * E_PER_W + es
        eb = e // 8
        k = e % 8
        pltpu.sync_copy(tableT_hbm.at[e], erow_v)
        for slot in range(XRING):
            pltpu.async_copy(xT_hbm.at[slot], xr_v.at[slot], xsem.at[slot])

        def fbody(f4, carry2):
            for sub in range(XRING):
                f = f4 * XRING + sub
                oslot = sub % 2
                pred = (
                    jnp.logical_or(es > 0, f4 >= 1) if sub < 2 else None
                )
                do_field(f, e, eb, k, sub, oslot, pred)
            return carry2

        lax.fori_loop(0, FIELDS // XRING, fbody, 0)
        for sub in range(FIELDS % XRING):
            f_tail = FIELDS - (FIELDS % XRING) + sub
            do_field(f_tail, e, eb, k, sub, sub % 2, None)
        return carry

    lax.fori_loop(0, E_PER_W, ebody, 0)
    owait(0)
    owait(1)


def kernel(x, table):
    outQ = _emb_gather(x.T, table.T)
    return outQ.transpose(2, 4, 0, 1, 3).reshape(BATCH, FIELDS, EMB)
